# Initial kernel scaffold; baseline (speedup 1.0000x reference)
import jax
import jax.numpy as jnp
from jax.experimental import pallas as pl


def kernel(batch, x, edge_index, beta, edge_attr, edge_weight, W_enc, b_enc, W1, b1, W2, b2):
    raise NotImplementedError("write your pallas kernel here")


# speedup vs baseline: 3.8196x; 3.8196x over previous
import functools

import jax
import jax.numpy as jnp
from jax import lax
from jax.experimental import pallas as pl
from jax.experimental.pallas import tpu as pltpu
from jax.experimental.pallas import tpu_sc as plsc

NC = 2
NS = 16
LN = 16
NW = NC * NS

CH = 80


def _tc_encode(x, W_enc, b_enc):
    def body(x_ref, w_ref, b_ref, o_ref):
        o_ref[...] = (
            jnp.dot(x_ref[...], w_ref[...], preferred_element_type=jnp.float32)
            + b_ref[...]
        )

    return pl.pallas_call(
        body,
        out_shape=jax.ShapeDtypeStruct(x.shape, jnp.float32),
    )(x, W_enc, b_enc.reshape(1, -1))


def _tc_node_mlp(p, h, beta, W1a, W1b, b1):
    n, d = h.shape
    hid = W1a.shape[1]

    def body(p_ref, h_ref, beta_ref, wa_ref, wb_ref, b1_ref, ab_ref):
        ne = jnp.maximum(p_ref[0] + p_ref[1] + beta_ref[0] * h_ref[...], 0.0)
        a = jnp.dot(ne, wa_ref[...], preferred_element_type=jnp.float32) + b1_ref[...]
        b = jnp.dot(ne, wb_ref[...], preferred_element_type=jnp.float32)
        ab_ref[...] = jnp.concatenate([a, b], axis=1)

    return pl.pallas_call(
        body,
        in_specs=[
            pl.BlockSpec(memory_space=pltpu.VMEM),
            pl.BlockSpec(memory_space=pltpu.VMEM),
            pl.BlockSpec(memory_space=pltpu.SMEM),
            pl.BlockSpec(memory_space=pltpu.VMEM),
            pl.BlockSpec(memory_space=pltpu.VMEM),
            pl.BlockSpec(memory_space=pltpu.VMEM),
        ],
        out_shape=jax.ShapeDtypeStruct((n, 2 * hid), jnp.float32),
    )(p, h, beta, W1a, W1b, b1.reshape(1, -1))


def _sc_aggregate(h, src, dst, ew, zeros):
    n, d = h.shape
    e = src.shape[0]
    epw = e // NW
    nchunk = epw // CH
    rows_per_tile = n // NS

    mesh = plsc.VectorSubcoreMesh(core_axis_name="c", subcore_axis_name="s")

    @functools.partial(
        pl.kernel,
        out_type=jax.ShapeDtypeStruct((NC, n, d), jnp.float32),
        mesh=mesh,
        compiler_params=pltpu.CompilerParams(needs_layout_passes=False),
        scratch_types=[
            pltpu.VMEM_SHARED((n, d), jnp.float32),
            pltpu.VMEM((CH,), jnp.int32),
            pltpu.VMEM((CH,), jnp.int32),
            pltpu.VMEM((CH,), jnp.float32),
            pltpu.VMEM((CH, d), jnp.float32),
            pltpu.SemaphoreType.DMA,
        ],
    )
    def k(h_hbm, src_hbm, dst_hbm, ew_hbm, z_hbm, part_hbm,
          acc_sh, sidx, didx, ewv, rows, sem):
        c = lax.axis_index("c")
        s = lax.axis_index("s")
        wid = c * NS + s
        r0 = s * rows_per_tile
        pltpu.sync_copy(z_hbm.at[pl.ds(r0, rows_per_tile)],
                        acc_sh.at[pl.ds(r0, rows_per_tile)])
        plsc.subcore_barrier()

        def chunk(kk, _):
            base = wid * epw + kk * CH
            pltpu.sync_copy(src_hbm.at[pl.ds(base, CH)], sidx)
            pltpu.sync_copy(ew_hbm.at[pl.ds(base, CH)], ewv)
            pltpu.async_copy(h_hbm.at[sidx], rows, sem).wait()

            def scale(i, _):
                splat = jnp.zeros((LN,), jnp.int32) + i
                w = plsc.load_gather(ewv, [splat])
                for r in range(d // LN):
                    rows[i, pl.ds(r * LN, LN)] = rows[i, pl.ds(r * LN, LN)] * w
                return _

            lax.fori_loop(0, CH, scale, 0, unroll=2)
            pltpu.sync_copy(dst_hbm.at[pl.ds(base, CH)], didx)
            pltpu.sync_copy(rows, acc_sh.at[didx], add=True)
            return _

        lax.fori_loop(0, nchunk, chunk, 0)
        plsc.subcore_barrier()
        pltpu.sync_copy(acc_sh.at[pl.ds(r0, rows_per_tile)],
                        part_hbm.at[c, pl.ds(r0, rows_per_tile)])

    return k(h, src, dst, ew, zeros)


def _sc_edge_logits(AB, src, dst, w2, b2):
    n, two_hid = AB.shape
    hid = two_hid // 2
    e = src.shape[0]
    epw = e // NW
    nchunk = epw // CH

    mesh = plsc.VectorSubcoreMesh(core_axis_name="c", subcore_axis_name="s")

    @functools.partial(
        pl.kernel,
        out_type=jax.ShapeDtypeStruct((e,), jnp.float32),
        mesh=mesh,
        compiler_params=pltpu.CompilerParams(needs_layout_passes=False),
        scratch_types=[
            pltpu.VMEM((CH,), jnp.int32),
            pltpu.VMEM((CH,), jnp.int32),
            pltpu.VMEM((CH, two_hid), jnp.float32),
            pltpu.VMEM((CH, two_hid), jnp.float32),
            pltpu.VMEM((CH,), jnp.float32),
            pltpu.VMEM((hid,), jnp.float32),
            pltpu.VMEM((16,), jnp.float32),
            pltpu.SemaphoreType.DMA,
            pltpu.SemaphoreType.DMA,
        ],
    )
    def k(ab_hbm, src_hbm, dst_hbm, w2_hbm, b2_hbm, out_hbm,
          sidx, didx, arows, brows, outv, w2v, b2v, sem_a, sem_b):
        c = lax.axis_index("c")
        s = lax.axis_index("s")
        wid = c * NS + s
        pltpu.sync_copy(w2_hbm, w2v)
        pltpu.sync_copy(b2_hbm, b2v)
        w2r = [w2v[pl.ds(r * LN, LN)] for r in range(hid // LN)]
        b2vec = b2v[pl.ds(0, LN)]
        lane = lax.iota(jnp.int32, LN)

        def chunk(kk, _):
            base = wid * epw + kk * CH
            pltpu.sync_copy(src_hbm.at[pl.ds(base, CH)], sidx)
            pltpu.sync_copy(dst_hbm.at[pl.ds(base, CH)], didx)
            ca = pltpu.async_copy(ab_hbm.at[sidx], arows, sem_a)
            cb = pltpu.async_copy(ab_hbm.at[didx], brows, sem_b)
            ca.wait()
            cb.wait()

            def group(g, _):
                acc = b2vec
                for j in range(LN):
                    i = g * LN + j
                    t = None
                    for r in range(hid // LN):
                        v = jnp.maximum(
                            arows[i, pl.ds(r * LN, LN)]
                            + brows[i, pl.ds(hid + r * LN, LN)],
                            0.0,
                        ) * w2r[r]
                        t = v if t is None else t + v
                    acc = jnp.where(lane == j, acc + jnp.sum(t), acc)
                outv[pl.ds(g * LN, LN)] = acc
                return _

            lax.fori_loop(0, CH // LN, group, 0)
            pltpu.sync_copy(outv, out_hbm.at[pl.ds(base, CH)])
            return _

        lax.fori_loop(0, nchunk, chunk, 0)

    return k(AB, src, dst, w2, b2)


def kernel(batch, x, edge_index, beta, edge_attr, edge_weight,
           W_enc, b_enc, W1, b1, W2, b2):
    n, d = x.shape
    e = edge_index.shape[1]
    src = edge_index[0]
    dst = edge_index[1]

    np_ = ((n + NS * 8 - 1) // (NS * 8)) * (NS * 8)
    x = jnp.pad(x, ((0, np_ - n), (0, 0)))

    h = _tc_encode(x, W_enc, b_enc)
    zeros = jnp.zeros((np_, d), dtype=jnp.float32)
    partials = _sc_aggregate(h, src, dst, edge_weight, zeros)
    AB = _tc_node_mlp(partials, h, beta, W1[:d], W1[d:], b1)
    b2pad = jnp.full((16,), b2[0], jnp.float32)
    logits = _sc_edge_logits(AB, src, dst, W2[:, 0], b2pad)
    return logits.reshape(e, 1)
